# Initial kernel scaffold; baseline (speedup 1.0000x reference)
import jax
import jax.numpy as jnp
from jax.experimental import pallas as pl


def kernel(x_nchw, w_oihw, gamma, beta):
    raise NotImplementedError("write your pallas kernel here")


# speedup vs baseline: 6.1068x; 6.1068x over previous
import jax
import jax.numpy as jnp
from jax.experimental import pallas as pl
from jax.experimental.pallas import tpu as pltpu


def _round_up(x, m):
    return ((x + m - 1) // m) * m


def _conv_stats_kernel(x_ref, b_ref, y_ref, s_ref, ss_ref):
    nb, hp, lanes = x_ref.shape
    h = hp - 2
    acc = jnp.dot(x_ref[:, 0:h, :].reshape(nb * h, lanes), b_ref[0],
                  preferred_element_type=jnp.float32)
    acc = acc + jnp.dot(x_ref[:, 1:h + 1, :].reshape(nb * h, lanes), b_ref[1],
                        preferred_element_type=jnp.float32)
    acc = acc + jnp.dot(x_ref[:, 2:h + 2, :].reshape(nb * h, lanes), b_ref[2],
                        preferred_element_type=jnp.float32)
    y_ref[...] = acc.astype(y_ref.dtype)
    s_ref[0] = jnp.sum(acc, axis=0, keepdims=True)
    ss_ref[0] = jnp.sum(acc * acc, axis=0, keepdims=True)


def _bn_relu_kernel(y_ref, scale_ref, shift_ref, o_ref):
    y = y_ref[...].astype(jnp.float32)
    o_ref[...] = jnp.maximum(y * scale_ref[...] + shift_ref[...], 0.0)


def kernel(x_nchw, w_oihw, gamma, beta, eps=1e-5):
    N, Cin, H, W = x_nchw.shape
    Cout, Cin2, KH, KW = w_oihw.shape
    assert Cin2 == Cin and KH == 3 and KW == 3

    Wp = W + 2
    lanes = Wp * Cin
    Lout = W * Cout
    LoutP = _round_up(Lout, 128)
    M = N * H

    x_nhwc = jnp.transpose(x_nchw, (0, 2, 3, 1))
    x_pad = jnp.pad(x_nhwc, ((0, 0), (1, 1), (1, 1), (0, 0)))
    x_rows = x_pad.reshape(N, H + 2, lanes).astype(jnp.bfloat16)

    wt = jnp.transpose(w_oihw, (2, 3, 1, 0)).astype(jnp.float32)
    dx = jnp.arange(KW)[:, None, None]
    xw = jnp.arange(Wp)[None, :, None]
    wv = jnp.arange(W)[None, None, :]
    S = (xw == wv + dx).astype(jnp.float32)
    B = jnp.einsum('apw,daco->dpcwo', S, wt).reshape(KH, lanes, Lout)
    if LoutP != Lout:
        B = jnp.pad(B, ((0, 0), (0, 0), (0, LoutP - Lout)))
    B = B.astype(jnp.bfloat16)

    nb = 8
    while N % nb:
        nb //= 2
    T = N // nb

    cparams = pltpu.CompilerParams(
        dimension_semantics=("parallel",),
        vmem_limit_bytes=96 * 1024 * 1024,
    )

    conv_y, psum, psumsq = pl.pallas_call(
        _conv_stats_kernel,
        grid=(T,),
        out_shape=(
            jax.ShapeDtypeStruct((M, LoutP), jnp.bfloat16),
            jax.ShapeDtypeStruct((T, 1, LoutP), jnp.float32),
            jax.ShapeDtypeStruct((T, 1, LoutP), jnp.float32),
        ),
        in_specs=[
            pl.BlockSpec((nb, H + 2, lanes), lambda i: (i, 0, 0)),
            pl.BlockSpec((KH, lanes, LoutP), lambda i: (0, 0, 0)),
        ],
        out_specs=(
            pl.BlockSpec((nb * H, LoutP), lambda i: (i, 0)),
            pl.BlockSpec((1, 1, LoutP), lambda i: (i, 0, 0)),
            pl.BlockSpec((1, 1, LoutP), lambda i: (i, 0, 0)),
        ),
        compiler_params=cparams,
        cost_estimate=pl.CostEstimate(
            flops=2 * M * KH * lanes * LoutP,
            transcendentals=0,
            bytes_accessed=2 * (N * (H + 2) * lanes + KH * lanes * LoutP
                               + M * LoutP) + 8 * T * LoutP,
        ),
    )(x_rows, B)

    count = N * H * W
    lane_sum = jnp.sum(psum, axis=(0, 1))[:Lout]
    lane_sumsq = jnp.sum(psumsq, axis=(0, 1))[:Lout]
    ch_sum = lane_sum.reshape(W, Cout).sum(axis=0)
    ch_sumsq = lane_sumsq.reshape(W, Cout).sum(axis=0)
    mean = ch_sum / count
    var = jnp.maximum(ch_sumsq / count - mean * mean, 0.0)
    inv_std = jax.lax.rsqrt(var + eps)
    scale_c = gamma.astype(jnp.float32) * inv_std
    shift_c = beta.astype(jnp.float32) - mean * scale_c
    scale_v = jnp.tile(scale_c, W).reshape(1, Lout)
    shift_v = jnp.tile(shift_c, W).reshape(1, Lout)
    if LoutP != Lout:
        scale_v = jnp.pad(scale_v, ((0, 0), (0, LoutP - Lout)))
        shift_v = jnp.pad(shift_v, ((0, 0), (0, LoutP - Lout)))

    rows2 = 512
    while M % rows2:
        rows2 //= 2
    out2d = pl.pallas_call(
        _bn_relu_kernel,
        grid=(M // rows2,),
        out_shape=jax.ShapeDtypeStruct((M, LoutP), jnp.float32),
        in_specs=[
            pl.BlockSpec((rows2, LoutP), lambda i: (i, 0)),
            pl.BlockSpec((1, LoutP), lambda i: (0, 0)),
            pl.BlockSpec((1, LoutP), lambda i: (0, 0)),
        ],
        out_specs=pl.BlockSpec((rows2, LoutP), lambda i: (i, 0)),
        compiler_params=cparams,
        cost_estimate=pl.CostEstimate(
            flops=2 * M * LoutP,
            transcendentals=0,
            bytes_accessed=6 * M * LoutP,
        ),
    )(conv_y, scale_v, shift_v)

    out = out2d[:, :Lout].reshape(N, H, W, Cout)
    return jnp.transpose(out, (0, 3, 1, 2))
